# baseline (speedup 1.0000x reference)
import functools

import jax
import jax.numpy as jnp
from jax import lax
from jax.experimental import pallas as pl
from jax.experimental.pallas import tpu as pltpu
from jax.experimental.pallas import tpu_sc as plsc

F32 = jnp.float32
NT = 16
KCH = 80
NB = 4
G = 2
CB = 16
ZB = 632
ACC_R = NT * ZB


def _zero_acc_slice(rows, acc, sid, wf):
  def zb(j, c):
    for cc in range(wf // 16):
      rows[j, pl.ds(cc * 16, 16)] = jnp.zeros((16,), F32)
    return c

  lax.fori_loop(0, KCH, zb, 0)
  for k in range(ZB // KCH):
    pltpu.sync_copy(rows, acc.at[pl.ds(sid * ZB + k * KCH, KCH)])
  rem = ZB % KCH
  if rem:
    pltpu.sync_copy(rows.at[pl.ds(0, rem)],
                    acc.at[pl.ds(sid * ZB + (ZB // KCH) * KCH, rem)])


def _gather_scatter_block(gather_start, rowbufs, acc, dstv, gsems, ssems):
  gd = [None] * NB
  sd = [None] * NB
  pending = [False] * NB
  for j in range(CB):
    p = j % NB
    if pending[p]:
      sd[p].wait()
      pending[p] = False
    gd[p] = gather_start(j, rowbufs[p], gsems[p])
    if j >= G:
      q = (j - G) % NB
      gd[q].wait()
      sd[q] = pltpu.async_copy(rowbufs[q], acc.at[dstv.at[j - G]], ssems[q],
                               add=True)
      pending[q] = True
  for m in range(max(CB - G, 0), CB):
    q = m % NB
    gd[q].wait()
    if pending[q]:
      sd[q].wait()
      pending[q] = False
    sd[q] = pltpu.async_copy(rowbufs[q], acc.at[dstv.at[m]], ssems[q],
                             add=True)
    pending[q] = True
  for b in range(NB):
    if pending[b]:
      sd[b].wait()


def _seg_sum_split(wf, n_chunks):
  mesh = plsc.VectorSubcoreMesh(core_axis_name="c", subcore_axis_name="s")
  out_sd = jax.ShapeDtypeStruct((ACC_R, wf), F32)

  n_blocks = n_chunks // CB

  @functools.partial(
      pl.kernel,
      out_type=(out_sd, out_sd),
      mesh=mesh,
      scratch_types=[
          pltpu.VMEM_SHARED((ACC_R, wf), F32),
          pltpu.VMEM((CB, KCH), jnp.int32),
          pltpu.VMEM((CB, KCH), jnp.int32),
      ] + [pltpu.VMEM((KCH, wf), F32)] * NB
        + [pltpu.SemaphoreType.DMA] * (2 * NB),
  )
  def k(h2, src3, dst3, oa, ob, acc, srcv, dstv, *bufsem):
    rowbufs = bufsem[:NB]
    gsems = bufsem[NB:2 * NB]
    ssems = bufsem[2 * NB:3 * NB]
    cid = lax.axis_index("c")
    sid = lax.axis_index("s")
    _zero_acc_slice(rowbufs[0], acc, sid, wf)
    plsc.subcore_barrier()
    htab = h2.at[cid]

    def blk(b, carry):
      pltpu.sync_copy(src3.at[sid, pl.ds(b * CB, CB)], srcv)
      pltpu.sync_copy(dst3.at[sid, pl.ds(b * CB, CB)], dstv)

      def gather_start(j, buf, sem):
        return pltpu.async_copy(htab.at[srcv.at[j]], buf, sem)

      _gather_scatter_block(gather_start, rowbufs, acc, dstv,
                            gsems, ssems)
      return carry

    lax.fori_loop(0, n_blocks, blk, 0)
    plsc.subcore_barrier()

    @pl.when(cid == 0)
    def _():
      pltpu.sync_copy(acc.at[pl.ds(sid * ZB, ZB)], oa.at[pl.ds(sid * ZB, ZB)])

    @pl.when(cid == 1)
    def _():
      pltpu.sync_copy(acc.at[pl.ds(sid * ZB, ZB)], ob.at[pl.ds(sid * ZB, ZB)])

  return k


def _seg_sum_dup(wf, n_chunks):
  mesh = plsc.VectorSubcoreMesh(core_axis_name="c", subcore_axis_name="s")
  out_sd = jax.ShapeDtypeStruct((ACC_R, wf), F32)

  n_blocks = n_chunks // CB

  @functools.partial(
      pl.kernel,
      out_type=(out_sd, out_sd),
      mesh=mesh,
      scratch_types=[
          pltpu.VMEM_SHARED((ACC_R, wf), F32),
          pltpu.VMEM((CB, KCH), jnp.int32),
          pltpu.VMEM((CB, KCH), jnp.int32),
      ] + [pltpu.VMEM((KCH, wf), F32)] * NB
        + [pltpu.SemaphoreType.DMA] * (2 * NB),
  )
  def k(h, src3, dst3, oa, ob, acc, srcv, dstv, *bufsem):
    rowbufs = bufsem[:NB]
    gsems = bufsem[NB:2 * NB]
    ssems = bufsem[2 * NB:3 * NB]
    cid = lax.axis_index("c")
    sid = lax.axis_index("s")
    wid = cid * NT + sid
    _zero_acc_slice(rowbufs[0], acc, sid, wf)
    plsc.subcore_barrier()

    def blk(b, carry):
      pltpu.sync_copy(src3.at[wid, pl.ds(b * CB, CB)], srcv)
      pltpu.sync_copy(dst3.at[wid, pl.ds(b * CB, CB)], dstv)

      def gather_start(j, buf, sem):
        return pltpu.async_copy(h.at[srcv.at[j]], buf, sem)

      _gather_scatter_block(gather_start, rowbufs, acc, dstv,
                            gsems, ssems)
      return carry

    lax.fori_loop(0, n_blocks, blk, 0)
    plsc.subcore_barrier()

    @pl.when(cid == 0)
    def _():
      pltpu.sync_copy(acc.at[pl.ds(sid * ZB, ZB)], oa.at[pl.ds(sid * ZB, ZB)])

    @pl.when(cid == 1)
    def _():
      pltpu.sync_copy(acc.at[pl.ds(sid * ZB, ZB)], ob.at[pl.ds(sid * ZB, ZB)])

  return k


def _stage1_sum(h, aa, ab, w1, b1):
  n, dh = h.shape
  hh = w1.shape[1]
  tb = 1000
  g = n // tb

  def body(h_r, aa_r, ab_r, w_r, b_r, x_r, st_r):
    i = pl.program_id(0)
    u = h_r[...] + aa_r[...] + ab_r[...]
    x = jnp.dot(u, w_r[...], preferred_element_type=F32) + b_r[...][None, :]
    x_r[...] = x

    @pl.when(i == 0)
    def _():
      st_r[...] = jnp.zeros_like(st_r)

    st_r[0:1, :] += jnp.sum(x, axis=0, keepdims=True)
    st_r[1:2, :] += jnp.sum(x * x, axis=0, keepdims=True)

  return pl.pallas_call(
      body,
      grid=(g,),
      in_specs=[
          pl.BlockSpec((tb, dh), lambda i: (i, 0)),
          pl.BlockSpec((tb, dh), lambda i: (i, 0)),
          pl.BlockSpec((tb, dh), lambda i: (i, 0)),
          pl.BlockSpec((dh, hh), lambda i: (0, 0)),
          pl.BlockSpec((hh,), lambda i: (0,)),
      ],
      out_specs=[
          pl.BlockSpec((tb, hh), lambda i: (i, 0)),
          pl.BlockSpec((8, hh), lambda i: (0, 0)),
      ],
      out_shape=[
          jax.ShapeDtypeStruct((n, hh), F32),
          jax.ShapeDtypeStruct((8, hh), F32),
      ],
  )(h, aa, ab, w1, b1)


def _stage1(ha, hb, aa, ab, w1, b1):
  n, wfh = ha.shape
  wfa = aa.shape[1]
  dh, hh = w1.shape
  tb = 1000
  g = n // tb

  def body(ha_r, hb_r, aa_r, ab_r, w_r, b_r, x_r, st_r):
    i = pl.program_id(0)
    u = jnp.concatenate([ha_r[...] + aa_r[...], hb_r[...] + ab_r[...]],
                        axis=1)
    x = jnp.dot(u, w_r[...], preferred_element_type=F32) + b_r[...][None, :]
    x_r[...] = x

    @pl.when(i == 0)
    def _():
      st_r[...] = jnp.zeros_like(st_r)

    st_r[0:1, :] += jnp.sum(x, axis=0, keepdims=True)
    st_r[1:2, :] += jnp.sum(x * x, axis=0, keepdims=True)

  return pl.pallas_call(
      body,
      grid=(g,),
      in_specs=[
          pl.BlockSpec((tb, wfh), lambda i: (i, 0)),
          pl.BlockSpec((tb, wfh), lambda i: (i, 0)),
          pl.BlockSpec((tb, wfa), lambda i: (i, 0)),
          pl.BlockSpec((tb, wfa), lambda i: (i, 0)),
          pl.BlockSpec((dh, hh), lambda i: (0, 0)),
          pl.BlockSpec((hh,), lambda i: (0,)),
      ],
      out_specs=[
          pl.BlockSpec((tb, hh), lambda i: (i, 0)),
          pl.BlockSpec((8, hh), lambda i: (0, 0)),
      ],
      out_shape=[
          jax.ShapeDtypeStruct((n, hh), F32),
          jax.ShapeDtypeStruct((8, hh), F32),
      ],
  )(ha, hb, aa, ab, w1, b1)


def _stage2(x1, stats, gmm, be, w2, b2, n_rows):
  n, hh = x1.shape
  half = hh // 2
  tb = 1000
  g = n // tb
  inv_n = 1.0 / n_rows

  def body(x_r, st_r, g_r, be_r, w_r, b_r, oa_r, ob_r):
    mean = st_r[0:1, :] * inv_n
    var = st_r[1:2, :] * inv_n - mean * mean
    inv = lax.rsqrt(var + 1e-5)
    a = jnp.maximum((x_r[...] - mean) * inv * g_r[...][None, :]
                    + be_r[...][None, :], 0.0)
    hn = jnp.dot(a, w_r[...], preferred_element_type=F32) + b_r[...][None, :]
    hn = jnp.maximum(hn, 0.0)
    oa_r[...] = hn[:, :half]
    ob_r[...] = hn[:, half:]

  return pl.pallas_call(
      body,
      grid=(g,),
      in_specs=[
          pl.BlockSpec((tb, hh), lambda i: (i, 0)),
          pl.BlockSpec((8, hh), lambda i: (0, 0)),
          pl.BlockSpec((hh,), lambda i: (0,)),
          pl.BlockSpec((hh,), lambda i: (0,)),
          pl.BlockSpec((hh, hh), lambda i: (0, 0)),
          pl.BlockSpec((hh,), lambda i: (0,)),
      ],
      out_specs=[
          pl.BlockSpec((tb, half), lambda i: (i, 0)),
          pl.BlockSpec((tb, half), lambda i: (i, 0)),
      ],
      out_shape=[
          jax.ShapeDtypeStruct((n, half), F32),
          jax.ShapeDtypeStruct((n, half), F32),
      ],
  )(x1, stats, gmm, be, w2, b2)


def _stage2_last(x1, stats, gmm, be, w2, b2, fc_w, fc_b, thr, n_rows):
  n, hh = x1.shape
  tb = 1000
  g = n // tb
  inv_n = 1.0 / n_rows

  def body(x_r, st_r, g_r, be_r, w_r, b_r, fw_r, fb_r, th_r, o_r):
    mean = st_r[0:1, :] * inv_n
    var = st_r[1:2, :] * inv_n - mean * mean
    inv = lax.rsqrt(var + 1e-5)
    a = jnp.maximum((x_r[...] - mean) * inv * g_r[...][None, :]
                    + be_r[...][None, :], 0.0)
    hn = jnp.dot(a, w_r[...], preferred_element_type=F32) + b_r[...][None, :]
    hn = jnp.maximum(hn, 0.0)
    y = jnp.dot(hn, fw_r[...], preferred_element_type=F32)
    o_r[...] = y + fb_r[...][None, :] - th_r[...]

  return pl.pallas_call(
      body,
      grid=(g,),
      in_specs=[
          pl.BlockSpec((tb, hh), lambda i: (i, 0)),
          pl.BlockSpec((8, hh), lambda i: (0, 0)),
          pl.BlockSpec((hh,), lambda i: (0,)),
          pl.BlockSpec((hh,), lambda i: (0,)),
          pl.BlockSpec((hh, hh), lambda i: (0, 0)),
          pl.BlockSpec((hh,), lambda i: (0,)),
          pl.BlockSpec((hh, 1), lambda i: (0, 0)),
          pl.BlockSpec((1,), lambda i: (0,)),
          pl.BlockSpec((1, 1), lambda i: (0, 0)),
      ],
      out_specs=pl.BlockSpec((tb, 1), lambda i: (i, 0)),
      out_shape=jax.ShapeDtypeStruct((n, 1), F32),
  )(x1, stats, gmm, be, w2, b2, fc_w, fc_b, thr)


def kernel(features, edge_index,
           W1_0, b1_0, g_0, be_0, W2_0, b2_0,
           W1_1, b1_1, g_1, be_1, W2_1, b2_1,
           W1_2, b1_2, g_2, be_2, W2_2, b2_2,
           fc_W, fc_b, cl_thres):
  n, d = features.shape
  e = edge_index.shape[1]
  params = [
      (W1_0, b1_0, g_0, be_0, W2_0, b2_0),
      (W1_1, b1_1, g_1, be_1, W2_1, b2_1),
      (W1_2, b1_2, g_2, be_2, W2_2, b2_2),
  ]

  src = edge_index[0]
  dst = edge_index[1]

  def chunked(ix, pad_val, n_workers):
    n_chunks = -(-e // (n_workers * KCH * CB)) * CB
    pad = n_workers * n_chunks * KCH - e
    return (jnp.concatenate([ix, jnp.full((pad,), pad_val, jnp.int32)])
            .reshape(n_workers, n_chunks, KCH), n_chunks)

  src3s, ncs = chunked(src, 0, NT)
  dst3s, _ = chunked(dst, n, NT)
  src3d, ncd = chunked(src, 0, 2 * NT)
  dst3d, _ = chunked(dst, n, 2 * NT)
  thr = cl_thres.reshape(1, 1)

  agg_a, agg_b = _seg_sum_dup(d, ncd)(features, src3d, dst3d)
  x1, stats = _stage1_sum(features, agg_a, agg_b, W1_0, b1_0)
  ha, hb = _stage2(x1, stats, g_0, be_0, W2_0, b2_0, n)

  for l in (1, 2):
    w1, b1, gmm, be, w2, b2 = params[l]
    wf = ha.shape[1]
    h2 = jnp.stack([ha, hb])
    agg_a, agg_b = _seg_sum_split(wf, ncs)(h2, src3s, dst3s)
    x1, stats = _stage1(ha, hb, agg_a, agg_b, w1, b1)
    if l < 2:
      ha, hb = _stage2(x1, stats, gmm, be, w2, b2, n)
    else:
      return _stage2_last(x1, stats, gmm, be, w2, b2, fc_W, fc_b, thr, n)


# speedup vs baseline: 1.1380x; 1.1380x over previous
import functools

import jax
import jax.numpy as jnp
from jax import lax
from jax.experimental import pallas as pl
from jax.experimental.pallas import tpu as pltpu
from jax.experimental.pallas import tpu_sc as plsc

F32 = jnp.float32
NT = 16
KCH = 64
NB = 5
G = 2
CB = 16
ZB = 632
ACC_R = NT * ZB


def _zero_acc_slice(rows, acc, sid, wf):
  def zb(j, c):
    for cc in range(wf // 16):
      rows[j, pl.ds(cc * 16, 16)] = jnp.zeros((16,), F32)
    return c

  lax.fori_loop(0, KCH, zb, 0)
  for k in range(ZB // KCH):
    pltpu.sync_copy(rows, acc.at[pl.ds(sid * ZB + k * KCH, KCH)])
  rem = ZB % KCH
  if rem:
    pltpu.sync_copy(rows.at[pl.ds(0, rem)],
                    acc.at[pl.ds(sid * ZB + (ZB // KCH) * KCH, rem)])


def _gather_scatter_block(gather_start, rowbufs, acc, dstv, gsems, ssems):
  gd = [None] * NB
  sd = [None] * NB
  pending = [False] * NB
  for j in range(CB):
    p = j % NB
    if pending[p]:
      sd[p].wait()
      pending[p] = False
    gd[p] = gather_start(j, rowbufs[p], gsems[p])
    if j >= G:
      q = (j - G) % NB
      gd[q].wait()
      sd[q] = pltpu.async_copy(rowbufs[q], acc.at[dstv.at[j - G]], ssems[q],
                               add=True)
      pending[q] = True
  for m in range(max(CB - G, 0), CB):
    q = m % NB
    gd[q].wait()
    if pending[q]:
      sd[q].wait()
      pending[q] = False
    sd[q] = pltpu.async_copy(rowbufs[q], acc.at[dstv.at[m]], ssems[q],
                             add=True)
    pending[q] = True
  for b in range(NB):
    if pending[b]:
      sd[b].wait()


def _seg_sum_split(wf, n_chunks):
  mesh = plsc.VectorSubcoreMesh(core_axis_name="c", subcore_axis_name="s")
  out_sd = jax.ShapeDtypeStruct((ACC_R, wf), F32)

  n_blocks = n_chunks // CB

  @functools.partial(
      pl.kernel,
      out_type=(out_sd, out_sd),
      mesh=mesh,
      scratch_types=[
          pltpu.VMEM_SHARED((ACC_R, wf), F32),
          pltpu.VMEM((CB, KCH), jnp.int32),
          pltpu.VMEM((CB, KCH), jnp.int32),
      ] + [pltpu.VMEM((KCH, wf), F32)] * NB
        + [pltpu.SemaphoreType.DMA] * (2 * NB),
  )
  def k(h2, src3, dst3, oa, ob, acc, srcv, dstv, *bufsem):
    rowbufs = bufsem[:NB]
    gsems = bufsem[NB:2 * NB]
    ssems = bufsem[2 * NB:3 * NB]
    cid = lax.axis_index("c")
    sid = lax.axis_index("s")
    _zero_acc_slice(rowbufs[0], acc, sid, wf)
    plsc.subcore_barrier()
    htab = h2.at[cid]

    def blk(b, carry):
      pltpu.sync_copy(src3.at[sid, pl.ds(b * CB, CB)], srcv)
      pltpu.sync_copy(dst3.at[sid, pl.ds(b * CB, CB)], dstv)

      def gather_start(j, buf, sem):
        return pltpu.async_copy(htab.at[srcv.at[j]], buf, sem)

      _gather_scatter_block(gather_start, rowbufs, acc, dstv,
                            gsems, ssems)
      return carry

    lax.fori_loop(0, n_blocks, blk, 0)
    plsc.subcore_barrier()

    @pl.when(cid == 0)
    def _():
      pltpu.sync_copy(acc.at[pl.ds(sid * ZB, ZB)], oa.at[pl.ds(sid * ZB, ZB)])

    @pl.when(cid == 1)
    def _():
      pltpu.sync_copy(acc.at[pl.ds(sid * ZB, ZB)], ob.at[pl.ds(sid * ZB, ZB)])

  return k


def _seg_sum_dup(wf, n_chunks):
  mesh = plsc.VectorSubcoreMesh(core_axis_name="c", subcore_axis_name="s")
  out_sd = jax.ShapeDtypeStruct((ACC_R, wf), F32)

  n_blocks = n_chunks // CB

  @functools.partial(
      pl.kernel,
      out_type=(out_sd, out_sd),
      mesh=mesh,
      scratch_types=[
          pltpu.VMEM_SHARED((ACC_R, wf), F32),
          pltpu.VMEM((CB, KCH), jnp.int32),
          pltpu.VMEM((CB, KCH), jnp.int32),
      ] + [pltpu.VMEM((KCH, wf), F32)] * NB
        + [pltpu.SemaphoreType.DMA] * (2 * NB),
  )
  def k(h, src3, dst3, oa, ob, acc, srcv, dstv, *bufsem):
    rowbufs = bufsem[:NB]
    gsems = bufsem[NB:2 * NB]
    ssems = bufsem[2 * NB:3 * NB]
    cid = lax.axis_index("c")
    sid = lax.axis_index("s")
    wid = cid * NT + sid
    _zero_acc_slice(rowbufs[0], acc, sid, wf)
    plsc.subcore_barrier()

    def blk(b, carry):
      pltpu.sync_copy(src3.at[wid, pl.ds(b * CB, CB)], srcv)
      pltpu.sync_copy(dst3.at[wid, pl.ds(b * CB, CB)], dstv)

      def gather_start(j, buf, sem):
        return pltpu.async_copy(h.at[srcv.at[j]], buf, sem)

      _gather_scatter_block(gather_start, rowbufs, acc, dstv,
                            gsems, ssems)
      return carry

    lax.fori_loop(0, n_blocks, blk, 0)
    plsc.subcore_barrier()

    @pl.when(cid == 0)
    def _():
      pltpu.sync_copy(acc.at[pl.ds(sid * ZB, ZB)], oa.at[pl.ds(sid * ZB, ZB)])

    @pl.when(cid == 1)
    def _():
      pltpu.sync_copy(acc.at[pl.ds(sid * ZB, ZB)], ob.at[pl.ds(sid * ZB, ZB)])

  return k


def _stage1_sum(h, aa, ab, w1, b1):
  n, dh = h.shape
  hh = w1.shape[1]
  tb = 1000
  g = n // tb

  def body(h_r, aa_r, ab_r, w_r, b_r, x_r, st_r):
    i = pl.program_id(0)
    u = h_r[...] + aa_r[...] + ab_r[...]
    x = jnp.dot(u, w_r[...], preferred_element_type=F32) + b_r[...][None, :]
    x_r[...] = x

    @pl.when(i == 0)
    def _():
      st_r[...] = jnp.zeros_like(st_r)

    st_r[0:1, :] += jnp.sum(x, axis=0, keepdims=True)
    st_r[1:2, :] += jnp.sum(x * x, axis=0, keepdims=True)

  return pl.pallas_call(
      body,
      grid=(g,),
      in_specs=[
          pl.BlockSpec((tb, dh), lambda i: (i, 0)),
          pl.BlockSpec((tb, dh), lambda i: (i, 0)),
          pl.BlockSpec((tb, dh), lambda i: (i, 0)),
          pl.BlockSpec((dh, hh), lambda i: (0, 0)),
          pl.BlockSpec((hh,), lambda i: (0,)),
      ],
      out_specs=[
          pl.BlockSpec((tb, hh), lambda i: (i, 0)),
          pl.BlockSpec((8, hh), lambda i: (0, 0)),
      ],
      out_shape=[
          jax.ShapeDtypeStruct((n, hh), F32),
          jax.ShapeDtypeStruct((8, hh), F32),
      ],
  )(h, aa, ab, w1, b1)


def _stage1(ha, hb, aa, ab, w1, b1):
  n, wfh = ha.shape
  wfa = aa.shape[1]
  dh, hh = w1.shape
  tb = 1000
  g = n // tb

  def body(ha_r, hb_r, aa_r, ab_r, w_r, b_r, x_r, st_r):
    i = pl.program_id(0)
    u = jnp.concatenate([ha_r[...] + aa_r[...], hb_r[...] + ab_r[...]],
                        axis=1)
    x = jnp.dot(u, w_r[...], preferred_element_type=F32) + b_r[...][None, :]
    x_r[...] = x

    @pl.when(i == 0)
    def _():
      st_r[...] = jnp.zeros_like(st_r)

    st_r[0:1, :] += jnp.sum(x, axis=0, keepdims=True)
    st_r[1:2, :] += jnp.sum(x * x, axis=0, keepdims=True)

  return pl.pallas_call(
      body,
      grid=(g,),
      in_specs=[
          pl.BlockSpec((tb, wfh), lambda i: (i, 0)),
          pl.BlockSpec((tb, wfh), lambda i: (i, 0)),
          pl.BlockSpec((tb, wfa), lambda i: (i, 0)),
          pl.BlockSpec((tb, wfa), lambda i: (i, 0)),
          pl.BlockSpec((dh, hh), lambda i: (0, 0)),
          pl.BlockSpec((hh,), lambda i: (0,)),
      ],
      out_specs=[
          pl.BlockSpec((tb, hh), lambda i: (i, 0)),
          pl.BlockSpec((8, hh), lambda i: (0, 0)),
      ],
      out_shape=[
          jax.ShapeDtypeStruct((n, hh), F32),
          jax.ShapeDtypeStruct((8, hh), F32),
      ],
  )(ha, hb, aa, ab, w1, b1)


def _stage2(x1, stats, gmm, be, w2, b2, n_rows):
  n, hh = x1.shape
  half = hh // 2
  tb = 1000
  g = n // tb
  inv_n = 1.0 / n_rows

  def body(x_r, st_r, g_r, be_r, w_r, b_r, oa_r, ob_r):
    mean = st_r[0:1, :] * inv_n
    var = st_r[1:2, :] * inv_n - mean * mean
    inv = lax.rsqrt(var + 1e-5)
    a = jnp.maximum((x_r[...] - mean) * inv * g_r[...][None, :]
                    + be_r[...][None, :], 0.0)
    hn = jnp.dot(a, w_r[...], preferred_element_type=F32) + b_r[...][None, :]
    hn = jnp.maximum(hn, 0.0)
    oa_r[...] = hn[:, :half]
    ob_r[...] = hn[:, half:]

  return pl.pallas_call(
      body,
      grid=(g,),
      in_specs=[
          pl.BlockSpec((tb, hh), lambda i: (i, 0)),
          pl.BlockSpec((8, hh), lambda i: (0, 0)),
          pl.BlockSpec((hh,), lambda i: (0,)),
          pl.BlockSpec((hh,), lambda i: (0,)),
          pl.BlockSpec((hh, hh), lambda i: (0, 0)),
          pl.BlockSpec((hh,), lambda i: (0,)),
      ],
      out_specs=[
          pl.BlockSpec((tb, half), lambda i: (i, 0)),
          pl.BlockSpec((tb, half), lambda i: (i, 0)),
      ],
      out_shape=[
          jax.ShapeDtypeStruct((n, half), F32),
          jax.ShapeDtypeStruct((n, half), F32),
      ],
  )(x1, stats, gmm, be, w2, b2)


def _stage2_last(x1, stats, gmm, be, w2, b2, fc_w, fc_b, thr, n_rows):
  n, hh = x1.shape
  tb = 1000
  g = n // tb
  inv_n = 1.0 / n_rows

  def body(x_r, st_r, g_r, be_r, w_r, b_r, fw_r, fb_r, th_r, o_r):
    mean = st_r[0:1, :] * inv_n
    var = st_r[1:2, :] * inv_n - mean * mean
    inv = lax.rsqrt(var + 1e-5)
    a = jnp.maximum((x_r[...] - mean) * inv * g_r[...][None, :]
                    + be_r[...][None, :], 0.0)
    hn = jnp.dot(a, w_r[...], preferred_element_type=F32) + b_r[...][None, :]
    hn = jnp.maximum(hn, 0.0)
    y = jnp.dot(hn, fw_r[...], preferred_element_type=F32)
    o_r[...] = y + fb_r[...][None, :] - th_r[...]

  return pl.pallas_call(
      body,
      grid=(g,),
      in_specs=[
          pl.BlockSpec((tb, hh), lambda i: (i, 0)),
          pl.BlockSpec((8, hh), lambda i: (0, 0)),
          pl.BlockSpec((hh,), lambda i: (0,)),
          pl.BlockSpec((hh,), lambda i: (0,)),
          pl.BlockSpec((hh, hh), lambda i: (0, 0)),
          pl.BlockSpec((hh,), lambda i: (0,)),
          pl.BlockSpec((hh, 1), lambda i: (0, 0)),
          pl.BlockSpec((1,), lambda i: (0,)),
          pl.BlockSpec((1, 1), lambda i: (0, 0)),
      ],
      out_specs=pl.BlockSpec((tb, 1), lambda i: (i, 0)),
      out_shape=jax.ShapeDtypeStruct((n, 1), F32),
  )(x1, stats, gmm, be, w2, b2, fc_w, fc_b, thr)


def kernel(features, edge_index,
           W1_0, b1_0, g_0, be_0, W2_0, b2_0,
           W1_1, b1_1, g_1, be_1, W2_1, b2_1,
           W1_2, b1_2, g_2, be_2, W2_2, b2_2,
           fc_W, fc_b, cl_thres):
  n, d = features.shape
  e = edge_index.shape[1]
  params = [
      (W1_0, b1_0, g_0, be_0, W2_0, b2_0),
      (W1_1, b1_1, g_1, be_1, W2_1, b2_1),
      (W1_2, b1_2, g_2, be_2, W2_2, b2_2),
  ]

  src = edge_index[0]
  dst = edge_index[1]

  def chunked(ix, pad_val, n_workers):
    n_chunks = -(-e // (n_workers * KCH * CB)) * CB
    pad = n_workers * n_chunks * KCH - e
    return (jnp.concatenate([ix, jnp.full((pad,), pad_val, jnp.int32)])
            .reshape(n_workers, n_chunks, KCH), n_chunks)

  src3s, ncs = chunked(src, 0, NT)
  dst3s, _ = chunked(dst, n, NT)
  src3d, ncd = chunked(src, 0, 2 * NT)
  dst3d, _ = chunked(dst, n, 2 * NT)
  thr = cl_thres.reshape(1, 1)

  agg_a, agg_b = _seg_sum_dup(d, ncd)(features, src3d, dst3d)
  x1, stats = _stage1_sum(features, agg_a, agg_b, W1_0, b1_0)
  ha, hb = _stage2(x1, stats, g_0, be_0, W2_0, b2_0, n)

  for l in (1, 2):
    w1, b1, gmm, be, w2, b2 = params[l]
    wf = ha.shape[1]
    h2 = jnp.stack([ha, hb])
    agg_a, agg_b = _seg_sum_split(wf, ncs)(h2, src3s, dst3s)
    x1, stats = _stage1(ha, hb, agg_a, agg_b, w1, b1)
    if l < 2:
      ha, hb = _stage2(x1, stats, gmm, be, w2, b2, n)
    else:
      return _stage2_last(x1, stats, gmm, be, w2, b2, fc_W, fc_b, thr, n)
